# Initial kernel scaffold; baseline (speedup 1.0000x reference)
import jax
import jax.numpy as jnp
from jax.experimental import pallas as pl


def kernel(x, edge_index, edge_attr, Wl1, bl1, Wr1, br1, We1, att1, bias1, Wl2, bl2, Wr2, br2, We2, att2, bias2, Wo1, bo1, Wo2, bo2):
    raise NotImplementedError("write your pallas kernel here")


# speedup vs baseline: 14.6740x; 14.6740x over previous
import functools

import jax
import jax.numpy as jnp
import numpy as np
from jax import lax
from jax.experimental import pallas as pl
from jax.experimental.pallas import tpu as pltpu
from jax.experimental.pallas import tpu_sc as plsc

NN = 10000
EE = 320000
H = 5
C = 32
HC = 160
ED = 16
NEG = 0.2

NC = 2
NS = 16
NW = NC * NS
EPT = EE // NW
BC = 80
NCH = EPT // BC
BCB = 40
NCHB = EPT // BCB

_seg = np.zeros((HC, 16), np.float32)
for _h in range(H):
    _seg[_h * C:(_h + 1) * C, _h] = 1.0
SEG16 = _seg
SEGT16 = np.ascontiguousarray(_seg.T)

_mesh = plsc.VectorSubcoreMesh(core_axis_name="c", subcore_axis_name="s")


def _zero_rows(buf, nrows, width):
    zv = jnp.zeros((16,), jnp.float32)

    def body(i, _):
        for r in range(width // 16):
            buf[i, pl.ds(r * 16, 16)] = zv
        return 0

    lax.fori_loop(0, nrows, body, 0)


def _zero_shared(shared, zbuf, width, sid, bc):
    _zero_rows(zbuf, bc, width)

    def body(j, _):
        @pl.when(j % NS == sid)
        def _():
            pltpu.sync_copy(zbuf, shared.at[pl.ds(j * bc, bc)])
        return 0

    lax.fori_loop(0, NN // bc, body, 0)


@functools.partial(
    pl.kernel,
    out_type=jax.ShapeDtypeStruct((NC, NN, ED), jnp.float32),
    mesh=_mesh,
    compiler_params=pltpu.CompilerParams(use_tc_tiling_on_sc=False,
                                         needs_layout_passes=False),
    scratch_types=[
        pltpu.VMEM((BC,), jnp.int32),
        pltpu.VMEM((BC, ED), jnp.float32),
        pltpu.VMEM_SHARED((NN, ED), jnp.float32),
    ],
)
def _sc_esum(dst_hbm, ea_hbm, out_hbm, idx_d, arows, shared):
    cid = lax.axis_index("c")
    sid = lax.axis_index("s")
    wid = sid * NC + cid
    base = wid * EPT

    _zero_shared(shared, arows, ED, sid, BC)
    plsc.subcore_barrier()

    def chunk(j, _):
        off = base + j * BC
        pltpu.sync_copy(dst_hbm.at[pl.ds(off, BC)], idx_d)
        pltpu.sync_copy(ea_hbm.at[pl.ds(off, BC)], arows)
        pltpu.sync_copy(arows, shared.at[idx_d], add=True)
        return 0

    lax.fori_loop(0, NCH, chunk, 0)
    plsc.subcore_barrier()

    @pl.when(sid == 0)
    def _():
        pltpu.sync_copy(shared, out_hbm.at[cid])


@functools.partial(
    pl.kernel,
    out_type=(
        jax.ShapeDtypeStruct((EE, 16), jnp.float32),
        jax.ShapeDtypeStruct((NC, NN, 16), jnp.float32),
    ),
    mesh=_mesh,
    compiler_params=pltpu.CompilerParams(use_tc_tiling_on_sc=False,
                                         needs_layout_passes=False),
    scratch_types=[
        pltpu.VMEM((BC,), jnp.int32),
        pltpu.VMEM((BC,), jnp.int32),
        pltpu.VMEM((BC, HC), jnp.float32),
        pltpu.VMEM((BC, HC), jnp.float32),
        pltpu.VMEM((BC, HC), jnp.float32),
        pltpu.VMEM((BC, 16), jnp.float32),
        pltpu.VMEM((HC,), jnp.float32),
        pltpu.VMEM_SHARED((NN, 16), jnp.float32),
        pltpu.SemaphoreType.DMA,
        pltpu.SemaphoreType.DMA,
    ],
)
def _sc_pass_a(src_hbm, dst_hbm, xl_hbm, xr_hbm, ee_hbm, att_hbm,
               lg_hbm, sum_hbm,
               idx_s, idx_d, xlr, xrr, eer, lbuf, attb, shared, sem1, sem2):
    cid = lax.axis_index("c")
    sid = lax.axis_index("s")
    wid = sid * NC + cid
    base = wid * EPT

    _zero_shared(shared, lbuf, 16, sid, BC)
    plsc.subcore_barrier()

    pltpu.sync_copy(att_hbm, attb)
    lanes = lax.iota(jnp.int32, 16)
    ohs = [(lanes == h).astype(jnp.float32) for h in range(H)]
    c5 = (lanes == H).astype(jnp.float32)

    def chunk(j, _):
        off = base + j * BC
        pltpu.sync_copy(src_hbm.at[pl.ds(off, BC)], idx_s)
        pltpu.sync_copy(dst_hbm.at[pl.ds(off, BC)], idx_d)
        pltpu.sync_copy(xl_hbm.at[idx_s], xlr)
        pltpu.sync_copy(xr_hbm.at[idx_d], xrr)
        pltpu.sync_copy(ee_hbm.at[pl.ds(off, BC)], eer)

        def edge(e, _):
            lv = c5
            for h in range(H):
                acc = None
                for r in (2 * h, 2 * h + 1):
                    sl = pl.ds(r * 16, 16)
                    a = xlr[e, sl] + xrr[e, sl] + eer[e, sl]
                    t = jnp.maximum(a, 0.0) + NEG * jnp.minimum(a, 0.0)
                    p = t * attb[sl]
                    acc = p if acc is None else acc + p
                lv = lv + jnp.sum(acc) * ohs[h]
            lbuf[e, :] = lv
            return 0

        lax.fori_loop(0, BC, edge, 0)
        pltpu.sync_copy(lbuf, lg_hbm.at[pl.ds(off, BC)])
        pltpu.sync_copy(lbuf, shared.at[idx_d], add=True)
        return 0

    lax.fori_loop(0, NCH, chunk, 0)
    plsc.subcore_barrier()

    @pl.when(sid == 0)
    def _():
        pltpu.sync_copy(shared, sum_hbm.at[cid])


@functools.partial(
    pl.kernel,
    out_type=jax.ShapeDtypeStruct((NC, NN, HC + 16), jnp.float32),
    mesh=_mesh,
    compiler_params=pltpu.CompilerParams(use_tc_tiling_on_sc=False,
                                         needs_layout_passes=False),
    scratch_types=[
        pltpu.VMEM((BCB,), jnp.int32),
        pltpu.VMEM((BCB,), jnp.int32),
        pltpu.VMEM((BCB, HC), jnp.float32),
        pltpu.VMEM((BCB, 16), jnp.float32),
        pltpu.VMEM((BCB, 16), jnp.float32),
        pltpu.VMEM((BCB, HC + 16), jnp.float32),
        pltpu.VMEM_SHARED((NN, HC + 16), jnp.float32),
        pltpu.SemaphoreType.DMA,
        pltpu.SemaphoreType.DMA,
    ],
)
def _sc_pass_b(src_hbm, dst_hbm, lg_hbm, mbar_hbm, xl_hbm, out_hbm,
               idx_s, idx_d, xlr, lr, mr, ob, shared, sem1, sem2):
    cid = lax.axis_index("c")
    sid = lax.axis_index("s")
    wid = sid * NC + cid
    base = wid * EPT

    _zero_shared(shared, ob, HC + 16, sid, BCB)
    plsc.subcore_barrier()

    lanes = lax.iota(jnp.int32, 16)
    ohs = [(lanes == h).astype(jnp.float32) for h in range(H)]
    msk = (lanes < H).astype(jnp.float32)

    def chunk(j, _):
        off = base + j * BCB
        pltpu.sync_copy(src_hbm.at[pl.ds(off, BCB)], idx_s)
        pltpu.sync_copy(dst_hbm.at[pl.ds(off, BCB)], idx_d)
        pltpu.sync_copy(xl_hbm.at[idx_s], xlr)
        pltpu.sync_copy(mbar_hbm.at[idx_d], mr)
        pltpu.sync_copy(lg_hbm.at[pl.ds(off, BCB)], lr)

        def edge(e, _):
            ev = jnp.exp(lr[e, :] - mr[e, :]) * msk
            ob[e, pl.ds(HC, 16)] = ev
            for h in range(H):
                sh = jnp.sum(ev * ohs[h])
                for r in (2 * h, 2 * h + 1):
                    sl = pl.ds(r * 16, 16)
                    ob[e, sl] = xlr[e, sl] * sh
            return 0

        lax.fori_loop(0, BCB, edge, 0)
        pltpu.sync_copy(ob, shared.at[idx_d], add=True)
        return 0

    lax.fori_loop(0, NCHB, chunk, 0)
    plsc.subcore_barrier()

    @pl.when(sid == 0)
    def _():
        pltpu.sync_copy(shared, out_hbm.at[cid])


def _mm_bias(xarr, W, b, bm):
    M, K = xarr.shape
    Nout = W.shape[1]

    def body(x_ref, w_ref, b_ref, o_ref):
        o_ref[...] = jnp.dot(x_ref[...], w_ref[...],
                             preferred_element_type=jnp.float32) + b_ref[...]

    return pl.pallas_call(
        body,
        grid=(M // bm,),
        in_specs=[
            pl.BlockSpec((bm, K), lambda i: (i, 0)),
            pl.BlockSpec((K, Nout), lambda i: (0, 0)),
            pl.BlockSpec((1, Nout), lambda i: (0, 0)),
        ],
        out_specs=pl.BlockSpec((bm, Nout), lambda i: (i, 0)),
        out_shape=jax.ShapeDtypeStruct((M, Nout), jnp.float32),
    )(xarr, W, b.reshape(1, -1))


def _tc_prep(sl0, sl1, es0, es1, xl, xr, We, att_row, bm=400):
    def body(sl0_ref, sl1_ref, es0_ref, es1_ref, xl_ref, xr_ref, we_ref,
             att_ref, seg_ref, mbar_ref, elp_ref):
        slt = sl0_ref[...] + sl1_ref[...]
        est = es0_ref[...] + es1_ref[...]
        cnt = slt[:, H:H + 1]
        loop_attr = est / jnp.maximum(cnt, 1.0)
        lee = jnp.dot(loop_attr, we_ref[...], preferred_element_type=jnp.float32)
        a = xl_ref[...] + xr_ref[...] + lee
        t = jnp.maximum(a, 0.0) + NEG * jnp.minimum(a, 0.0)
        ll = jnp.dot(t * att_ref[...], seg_ref[...],
                     preferred_element_type=jnp.float32)
        mbar = (slt + ll) / (cnt + 1.0)
        mbar_ref[...] = mbar
        elp_ref[...] = jnp.exp(ll - mbar)

    n16 = pl.BlockSpec((bm, 16), lambda i: (i, 0))
    return pl.pallas_call(
        body,
        grid=(NN // bm,),
        in_specs=[
            n16, n16, n16, n16,
            pl.BlockSpec((bm, HC), lambda i: (i, 0)),
            pl.BlockSpec((bm, HC), lambda i: (i, 0)),
            pl.BlockSpec((ED, HC), lambda i: (0, 0)),
            pl.BlockSpec((1, HC), lambda i: (0, 0)),
            pl.BlockSpec((HC, 16), lambda i: (0, 0)),
        ],
        out_specs=[n16, n16],
        out_shape=[
            jax.ShapeDtypeStruct((NN, 16), jnp.float32),
            jax.ShapeDtypeStruct((NN, 16), jnp.float32),
        ],
    )(sl0, sl1, es0, es1, xl, xr, We, att_row, SEG16)


def _tc_epilogue(a0, a1, xl, elp, bias_row, bm=400):
    def body(a0_ref, a1_ref, xl_ref, elp_ref, b_ref, segt_ref, o_ref):
        acc = a0_ref[...] + a1_ref[...]
        elp = elp_ref[...]
        s16 = acc[:, HC:HC + 16] + elp
        den = jnp.dot(s16, segt_ref[...], preferred_element_type=jnp.float32)
        eexp = jnp.dot(elp, segt_ref[...], preferred_element_type=jnp.float32)
        num = acc[:, 0:HC] + xl_ref[...] * eexp
        o_ref[...] = jnp.maximum(num / (den + 1e-16) + b_ref[...], 0.0)

    return pl.pallas_call(
        body,
        grid=(NN // bm,),
        in_specs=[
            pl.BlockSpec((bm, HC + 16), lambda i: (i, 0)),
            pl.BlockSpec((bm, HC + 16), lambda i: (i, 0)),
            pl.BlockSpec((bm, HC), lambda i: (i, 0)),
            pl.BlockSpec((bm, 16), lambda i: (i, 0)),
            pl.BlockSpec((1, HC), lambda i: (0, 0)),
            pl.BlockSpec((16, HC), lambda i: (0, 0)),
        ],
        out_specs=pl.BlockSpec((bm, HC), lambda i: (i, 0)),
        out_shape=jax.ShapeDtypeStruct((NN, HC), jnp.float32),
    )(a0, a1, xl, elp, bias_row, SEGT16)


def _tc_final(h, Wo1, bo1, Wo2, bo2, bm=400):
    def body(h_ref, w1_ref, b1_ref, w2_ref, b2_ref, o_ref):
        t = jnp.dot(h_ref[...], w1_ref[...],
                    preferred_element_type=jnp.float32) + b1_ref[...]
        o_ref[...] = jnp.dot(t, w2_ref[...],
                             preferred_element_type=jnp.float32) + b2_ref[...]

    return pl.pallas_call(
        body,
        grid=(NN // bm,),
        in_specs=[
            pl.BlockSpec((bm, HC), lambda i: (i, 0)),
            pl.BlockSpec((HC, C), lambda i: (0, 0)),
            pl.BlockSpec((1, C), lambda i: (0, 0)),
            pl.BlockSpec((C, 64), lambda i: (0, 0)),
            pl.BlockSpec((1, 64), lambda i: (0, 0)),
        ],
        out_specs=pl.BlockSpec((bm, 64), lambda i: (i, 0)),
        out_shape=jax.ShapeDtypeStruct((NN, 64), jnp.float32),
    )(h, Wo1, bo1.reshape(1, -1), Wo2, bo2.reshape(1, -1))


def _conv(xin, src, dst, edge_attr, esum_p, Wl, bl, Wr, br, We, att, bias):
    xl = _mm_bias(xin, Wl, bl, 400)
    xr = _mm_bias(xin, Wr, br, 400)
    ee = _mm_bias(edge_attr, We, jnp.zeros((HC,), jnp.float32), 640)
    lg, slp = _sc_pass_a(src, dst, xl, xr, ee, att.reshape(-1))
    mbar, elp = _tc_prep(slp[0], slp[1], esum_p[0], esum_p[1], xl, xr,
                         We, att.reshape(1, -1))
    accp = _sc_pass_b(src, dst, lg, mbar, xl)
    return _tc_epilogue(accp[0], accp[1], xl, elp, bias.reshape(1, -1))


def kernel(x, edge_index, edge_attr,
           Wl1, bl1, Wr1, br1, We1, att1, bias1,
           Wl2, bl2, Wr2, br2, We2, att2, bias2,
           Wo1, bo1, Wo2, bo2):
    src = edge_index[0]
    dst = edge_index[1]
    esum_p = _sc_esum(dst, edge_attr)
    h1 = _conv(x, src, dst, edge_attr, esum_p,
               Wl1, bl1, Wr1, br1, We1, att1, bias1)
    h2 = _conv(h1, src, dst, edge_attr, esum_p,
               Wl2, bl2, Wr2, br2, We2, att2, bias2)
    return _tc_final(h2, Wo1, bo1, Wo2, bo2)


# speedup vs baseline: 18.6021x; 1.2677x over previous
import functools

import jax
import jax.numpy as jnp
import numpy as np
from jax import lax
from jax.experimental import pallas as pl
from jax.experimental.pallas import tpu as pltpu
from jax.experimental.pallas import tpu_sc as plsc

NN = 10000
EE = 320000
H = 5
C = 32
HC = 160
ED = 16
NEG = 0.2

NC = 2
NS = 16
NW = NC * NS
EPT = EE // NW
BC = 80
NCH = EPT // BC
BCB = 40
NCHB = EPT // BCB

_seg = np.zeros((HC, 16), np.float32)
for _h in range(H):
    _seg[_h * C:(_h + 1) * C, _h] = 1.0
SEG16 = _seg
SEGT16 = np.ascontiguousarray(_seg.T)

_mesh = plsc.VectorSubcoreMesh(core_axis_name="c", subcore_axis_name="s")


def _zero_rows(buf, nrows, width):
    zv = jnp.zeros((16,), jnp.float32)

    def body(i, _):
        for r in range(width // 16):
            buf[i, pl.ds(r * 16, 16)] = zv
        return 0

    lax.fori_loop(0, nrows, body, 0)


def _zero_shared(shared, zbuf, width, sid, bc):
    _zero_rows(zbuf, bc, width)

    def body(j, _):
        @pl.when(j % NS == sid)
        def _():
            pltpu.sync_copy(zbuf, shared.at[pl.ds(j * bc, bc)])
        return 0

    lax.fori_loop(0, NN // bc, body, 0)


@functools.partial(
    pl.kernel,
    out_type=jax.ShapeDtypeStruct((NC, NN, ED), jnp.float32),
    mesh=_mesh,
    compiler_params=pltpu.CompilerParams(use_tc_tiling_on_sc=False,
                                         needs_layout_passes=False),
    scratch_types=[
        pltpu.VMEM((BC,), jnp.int32),
        pltpu.VMEM((BC, ED), jnp.float32),
        pltpu.VMEM_SHARED((NN, ED), jnp.float32),
    ],
)
def _sc_esum(dst_hbm, ea_hbm, out_hbm, idx_d, arows, shared):
    cid = lax.axis_index("c")
    sid = lax.axis_index("s")
    wid = sid * NC + cid
    base = wid * EPT

    _zero_shared(shared, arows, ED, sid, BC)
    plsc.subcore_barrier()

    def chunk(j, _):
        off = base + j * BC
        pltpu.sync_copy(dst_hbm.at[pl.ds(off, BC)], idx_d)
        pltpu.sync_copy(ea_hbm.at[pl.ds(off, BC)], arows)
        pltpu.sync_copy(arows, shared.at[idx_d], add=True)
        return 0

    lax.fori_loop(0, NCH, chunk, 0)
    plsc.subcore_barrier()

    @pl.when(sid == 0)
    def _():
        pltpu.sync_copy(shared, out_hbm.at[cid])


@functools.partial(
    pl.kernel,
    out_type=(
        jax.ShapeDtypeStruct((EE, 16), jnp.float32),
        jax.ShapeDtypeStruct((NC, NN, 16), jnp.float32),
    ),
    mesh=_mesh,
    compiler_params=pltpu.CompilerParams(use_tc_tiling_on_sc=False,
                                         needs_layout_passes=False),
    scratch_types=[
        pltpu.VMEM((BC,), jnp.int32),
        pltpu.VMEM((BC,), jnp.int32),
        pltpu.VMEM((BC, HC), jnp.float32),
        pltpu.VMEM((BC, HC), jnp.float32),
        pltpu.VMEM((BC, HC), jnp.float32),
        pltpu.VMEM((BC, 16), jnp.float32),
        pltpu.VMEM((HC,), jnp.float32),
        pltpu.VMEM_SHARED((NN, 16), jnp.float32),
        pltpu.SemaphoreType.DMA,
        pltpu.SemaphoreType.DMA,
        pltpu.SemaphoreType.DMA,
    ],
)
def _sc_pass_a(src_hbm, dst_hbm, xl_hbm, xr_hbm, ee_hbm, att_hbm,
               lg_hbm, sum_hbm,
               idx_s, idx_d, xlr, xrr, eer, lbuf, attb, shared, sem1, sem2,
               sem3):
    cid = lax.axis_index("c")
    sid = lax.axis_index("s")
    wid = sid * NC + cid
    base = wid * EPT

    _zero_shared(shared, lbuf, 16, sid, BC)
    plsc.subcore_barrier()

    pltpu.sync_copy(att_hbm, attb)
    lanes = lax.iota(jnp.int32, 16)
    ohs = [(lanes == h).astype(jnp.float32) for h in range(H)]
    c5 = (lanes == H).astype(jnp.float32)

    def chunk(j, _):
        off = base + j * BC
        ci1 = pltpu.async_copy(src_hbm.at[pl.ds(off, BC)], idx_s, sem1)
        ci2 = pltpu.async_copy(dst_hbm.at[pl.ds(off, BC)], idx_d, sem2)
        cp3 = pltpu.async_copy(ee_hbm.at[pl.ds(off, BC)], eer, sem3)
        ci1.wait()
        ci2.wait()
        cp1 = pltpu.async_copy(xl_hbm.at[idx_s], xlr, sem1)
        cp2 = pltpu.async_copy(xr_hbm.at[idx_d], xrr, sem2)
        cp1.wait()
        cp2.wait()
        cp3.wait()

        def edge(e, _):
            lv = c5
            for h in range(H):
                acc = None
                for r in (2 * h, 2 * h + 1):
                    sl = pl.ds(r * 16, 16)
                    a = xlr[e, sl] + xrr[e, sl] + eer[e, sl]
                    t = jnp.maximum(a, 0.0) + NEG * jnp.minimum(a, 0.0)
                    p = t * attb[sl]
                    acc = p if acc is None else acc + p
                lv = lv + jnp.sum(acc) * ohs[h]
            lbuf[e, :] = lv
            return 0

        lax.fori_loop(0, BC, edge, 0)
        pltpu.sync_copy(lbuf, lg_hbm.at[pl.ds(off, BC)])
        pltpu.sync_copy(lbuf, shared.at[idx_d], add=True)
        return 0

    lax.fori_loop(0, NCH, chunk, 0)
    plsc.subcore_barrier()

    @pl.when(sid == 0)
    def _():
        pltpu.sync_copy(shared, sum_hbm.at[cid])


@functools.partial(
    pl.kernel,
    out_type=jax.ShapeDtypeStruct((NC, NN, HC + 16), jnp.float32),
    mesh=_mesh,
    compiler_params=pltpu.CompilerParams(use_tc_tiling_on_sc=False,
                                         needs_layout_passes=False),
    scratch_types=[
        pltpu.VMEM((BCB,), jnp.int32),
        pltpu.VMEM((BCB,), jnp.int32),
        pltpu.VMEM((BCB, HC), jnp.float32),
        pltpu.VMEM((BCB, 16), jnp.float32),
        pltpu.VMEM((BCB, 16), jnp.float32),
        pltpu.VMEM((BCB, HC + 16), jnp.float32),
        pltpu.VMEM_SHARED((NN, HC + 16), jnp.float32),
        pltpu.SemaphoreType.DMA,
        pltpu.SemaphoreType.DMA,
        pltpu.SemaphoreType.DMA,
    ],
)
def _sc_pass_b(src_hbm, dst_hbm, lg_hbm, mbar_hbm, xl_hbm, out_hbm,
               idx_s, idx_d, xlr, lr, mr, ob, shared, sem1, sem2, sem3):
    cid = lax.axis_index("c")
    sid = lax.axis_index("s")
    wid = sid * NC + cid
    base = wid * EPT

    _zero_shared(shared, ob, HC + 16, sid, BCB)
    plsc.subcore_barrier()

    lanes = lax.iota(jnp.int32, 16)
    ohs = [(lanes == h).astype(jnp.float32) for h in range(H)]
    msk = (lanes < H).astype(jnp.float32)

    def chunk(j, _):
        off = base + j * BCB
        ci1 = pltpu.async_copy(src_hbm.at[pl.ds(off, BCB)], idx_s, sem1)
        ci2 = pltpu.async_copy(dst_hbm.at[pl.ds(off, BCB)], idx_d, sem2)
        cp3 = pltpu.async_copy(lg_hbm.at[pl.ds(off, BCB)], lr, sem3)
        ci1.wait()
        ci2.wait()
        cp1 = pltpu.async_copy(xl_hbm.at[idx_s], xlr, sem1)
        cp2 = pltpu.async_copy(mbar_hbm.at[idx_d], mr, sem2)
        cp1.wait()
        cp2.wait()
        cp3.wait()

        def edge(e, _):
            ev = jnp.exp(lr[e, :] - mr[e, :]) * msk
            ob[e, pl.ds(HC, 16)] = ev
            for h in range(H):
                sh = jnp.sum(ev * ohs[h])
                for r in (2 * h, 2 * h + 1):
                    sl = pl.ds(r * 16, 16)
                    ob[e, sl] = xlr[e, sl] * sh
            return 0

        lax.fori_loop(0, BCB, edge, 0)
        pltpu.sync_copy(ob, shared.at[idx_d], add=True)
        return 0

    lax.fori_loop(0, NCHB, chunk, 0)
    plsc.subcore_barrier()

    @pl.when(sid == 0)
    def _():
        pltpu.sync_copy(shared, out_hbm.at[cid])


def _mm_bias(xarr, W, b, bm):
    M, K = xarr.shape
    Nout = W.shape[1]

    def body(x_ref, w_ref, b_ref, o_ref):
        o_ref[...] = jnp.dot(x_ref[...], w_ref[...],
                             preferred_element_type=jnp.float32) + b_ref[...]

    return pl.pallas_call(
        body,
        grid=(M // bm,),
        in_specs=[
            pl.BlockSpec((bm, K), lambda i: (i, 0)),
            pl.BlockSpec((K, Nout), lambda i: (0, 0)),
            pl.BlockSpec((1, Nout), lambda i: (0, 0)),
        ],
        out_specs=pl.BlockSpec((bm, Nout), lambda i: (i, 0)),
        out_shape=jax.ShapeDtypeStruct((M, Nout), jnp.float32),
    )(xarr, W, b.reshape(1, -1))


def _tc_prep(sl0, sl1, es0, es1, xl, xr, We, att_row, bm=400):
    def body(sl0_ref, sl1_ref, es0_ref, es1_ref, xl_ref, xr_ref, we_ref,
             att_ref, seg_ref, mbar_ref, elp_ref):
        slt = sl0_ref[...] + sl1_ref[...]
        est = es0_ref[...] + es1_ref[...]
        cnt = slt[:, H:H + 1]
        loop_attr = est / jnp.maximum(cnt, 1.0)
        lee = jnp.dot(loop_attr, we_ref[...], preferred_element_type=jnp.float32)
        a = xl_ref[...] + xr_ref[...] + lee
        t = jnp.maximum(a, 0.0) + NEG * jnp.minimum(a, 0.0)
        ll = jnp.dot(t * att_ref[...], seg_ref[...],
                     preferred_element_type=jnp.float32)
        mbar = (slt + ll) / (cnt + 1.0)
        mbar_ref[...] = mbar
        elp_ref[...] = jnp.exp(ll - mbar)

    n16 = pl.BlockSpec((bm, 16), lambda i: (i, 0))
    return pl.pallas_call(
        body,
        grid=(NN // bm,),
        in_specs=[
            n16, n16, n16, n16,
            pl.BlockSpec((bm, HC), lambda i: (i, 0)),
            pl.BlockSpec((bm, HC), lambda i: (i, 0)),
            pl.BlockSpec((ED, HC), lambda i: (0, 0)),
            pl.BlockSpec((1, HC), lambda i: (0, 0)),
            pl.BlockSpec((HC, 16), lambda i: (0, 0)),
        ],
        out_specs=[n16, n16],
        out_shape=[
            jax.ShapeDtypeStruct((NN, 16), jnp.float32),
            jax.ShapeDtypeStruct((NN, 16), jnp.float32),
        ],
    )(sl0, sl1, es0, es1, xl, xr, We, att_row, SEG16)


def _tc_epilogue(a0, a1, xl, elp, bias_row, bm=400):
    def body(a0_ref, a1_ref, xl_ref, elp_ref, b_ref, segt_ref, o_ref):
        acc = a0_ref[...] + a1_ref[...]
        elp = elp_ref[...]
        s16 = acc[:, HC:HC + 16] + elp
        den = jnp.dot(s16, segt_ref[...], preferred_element_type=jnp.float32)
        eexp = jnp.dot(elp, segt_ref[...], preferred_element_type=jnp.float32)
        num = acc[:, 0:HC] + xl_ref[...] * eexp
        o_ref[...] = jnp.maximum(num / (den + 1e-16) + b_ref[...], 0.0)

    return pl.pallas_call(
        body,
        grid=(NN // bm,),
        in_specs=[
            pl.BlockSpec((bm, HC + 16), lambda i: (i, 0)),
            pl.BlockSpec((bm, HC + 16), lambda i: (i, 0)),
            pl.BlockSpec((bm, HC), lambda i: (i, 0)),
            pl.BlockSpec((bm, 16), lambda i: (i, 0)),
            pl.BlockSpec((1, HC), lambda i: (0, 0)),
            pl.BlockSpec((16, HC), lambda i: (0, 0)),
        ],
        out_specs=pl.BlockSpec((bm, HC), lambda i: (i, 0)),
        out_shape=jax.ShapeDtypeStruct((NN, HC), jnp.float32),
    )(a0, a1, xl, elp, bias_row, SEGT16)


def _tc_final(h, Wo1, bo1, Wo2, bo2, bm=400):
    def body(h_ref, w1_ref, b1_ref, w2_ref, b2_ref, o_ref):
        t = jnp.dot(h_ref[...], w1_ref[...],
                    preferred_element_type=jnp.float32) + b1_ref[...]
        o_ref[...] = jnp.dot(t, w2_ref[...],
                             preferred_element_type=jnp.float32) + b2_ref[...]

    return pl.pallas_call(
        body,
        grid=(NN // bm,),
        in_specs=[
            pl.BlockSpec((bm, HC), lambda i: (i, 0)),
            pl.BlockSpec((HC, C), lambda i: (0, 0)),
            pl.BlockSpec((1, C), lambda i: (0, 0)),
            pl.BlockSpec((C, 64), lambda i: (0, 0)),
            pl.BlockSpec((1, 64), lambda i: (0, 0)),
        ],
        out_specs=pl.BlockSpec((bm, 64), lambda i: (i, 0)),
        out_shape=jax.ShapeDtypeStruct((NN, 64), jnp.float32),
    )(h, Wo1, bo1.reshape(1, -1), Wo2, bo2.reshape(1, -1))


def _conv(xin, src, dst, edge_attr, esum_p, Wl, bl, Wr, br, We, att, bias):
    xl = _mm_bias(xin, Wl, bl, 400)
    xr = _mm_bias(xin, Wr, br, 400)
    ee = _mm_bias(edge_attr, We, jnp.zeros((HC,), jnp.float32), 640)
    lg, slp = _sc_pass_a(src, dst, xl, xr, ee, att.reshape(-1))
    mbar, elp = _tc_prep(slp[0], slp[1], esum_p[0], esum_p[1], xl, xr,
                         We, att.reshape(1, -1))
    accp = _sc_pass_b(src, dst, lg, mbar, xl)
    return _tc_epilogue(accp[0], accp[1], xl, elp, bias.reshape(1, -1))


def kernel(x, edge_index, edge_attr,
           Wl1, bl1, Wr1, br1, We1, att1, bias1,
           Wl2, bl2, Wr2, br2, We2, att2, bias2,
           Wo1, bo1, Wo2, bo2):
    src = edge_index[0]
    dst = edge_index[1]
    esum_p = _sc_esum(dst, edge_attr)
    h1 = _conv(x, src, dst, edge_attr, esum_p,
               Wl1, bl1, Wr1, br1, We1, att1, bias1)
    h2 = _conv(h1, src, dst, edge_attr, esum_p,
               Wl2, bl2, Wr2, br2, We2, att2, bias2)
    return _tc_final(h2, Wo1, bo1, Wo2, bo2)


# speedup vs baseline: 21.8052x; 1.1722x over previous
import functools

import jax
import jax.numpy as jnp
import numpy as np
from jax import lax
from jax.experimental import pallas as pl
from jax.experimental.pallas import tpu as pltpu
from jax.experimental.pallas import tpu_sc as plsc

NN = 10000
EE = 320000
H = 5
C = 32
HC = 160
ED = 16
NEG = 0.2

NC = 2
NS = 16
NW = NC * NS
EPT = EE // NW
BC = 80
NCH = EPT // BC
BCB = 40
NCHB = EPT // BCB

_seg = np.zeros((HC, 16), np.float32)
for _h in range(H):
    _seg[_h * C:(_h + 1) * C, _h] = 1.0
SEG16 = _seg
SEGT16 = np.ascontiguousarray(_seg.T)

_mesh = plsc.VectorSubcoreMesh(core_axis_name="c", subcore_axis_name="s")


def _zero_rows(buf, nrows, width):
    zv = jnp.zeros((16,), jnp.float32)

    def body(i, _):
        for r in range(width // 16):
            buf[i, pl.ds(r * 16, 16)] = zv
        return 0

    lax.fori_loop(0, nrows, body, 0)


def _zero_shared(shared, zbuf, width, sid, bc):
    _zero_rows(zbuf, bc, width)

    def body(j, _):
        @pl.when(j % NS == sid)
        def _():
            pltpu.sync_copy(zbuf, shared.at[pl.ds(j * bc, bc)])
        return 0

    lax.fori_loop(0, NN // bc, body, 0)


@functools.partial(
    pl.kernel,
    out_type=jax.ShapeDtypeStruct((NC, NN, ED), jnp.float32),
    mesh=_mesh,
    compiler_params=pltpu.CompilerParams(use_tc_tiling_on_sc=False,
                                         needs_layout_passes=False),
    scratch_types=[
        pltpu.VMEM((BC,), jnp.int32),
        pltpu.VMEM((BC, ED), jnp.float32),
        pltpu.VMEM_SHARED((NN, ED), jnp.float32),
    ],
)
def _sc_esum(dst_hbm, ea_hbm, out_hbm, idx_d, arows, shared):
    cid = lax.axis_index("c")
    sid = lax.axis_index("s")
    wid = sid * NC + cid
    base = wid * EPT

    _zero_shared(shared, arows, ED, sid, BC)
    plsc.subcore_barrier()

    def chunk(j, _):
        off = base + j * BC
        pltpu.sync_copy(dst_hbm.at[pl.ds(off, BC)], idx_d)
        pltpu.sync_copy(ea_hbm.at[pl.ds(off, BC)], arows)
        pltpu.sync_copy(arows, shared.at[idx_d], add=True)
        return 0

    lax.fori_loop(0, NCH, chunk, 0)
    plsc.subcore_barrier()

    @pl.when(sid == 0)
    def _():
        pltpu.sync_copy(shared, out_hbm.at[cid])


@functools.partial(
    pl.kernel,
    out_type=(
        jax.ShapeDtypeStruct((EE, 16), jnp.float32),
        jax.ShapeDtypeStruct((NC, NN, 16), jnp.float32),
    ),
    mesh=_mesh,
    compiler_params=pltpu.CompilerParams(use_tc_tiling_on_sc=False,
                                         needs_layout_passes=False),
    scratch_types=[
        pltpu.VMEM((NCH, BC), jnp.int32),
        pltpu.VMEM((NCH, BC), jnp.int32),
        pltpu.VMEM((BC, HC), jnp.float32),
        pltpu.VMEM((BC, HC), jnp.float32),
        pltpu.VMEM((BC, HC), jnp.float32),
        pltpu.VMEM((BC, HC), jnp.float32),
        pltpu.VMEM((BC, HC), jnp.float32),
        pltpu.VMEM((BC, HC), jnp.float32),
        pltpu.VMEM((BC, 16), jnp.float32),
        pltpu.VMEM((HC,), jnp.float32),
        pltpu.VMEM_SHARED((NN, 16), jnp.float32),
        pltpu.SemaphoreType.DMA,
        pltpu.SemaphoreType.DMA,
        pltpu.SemaphoreType.DMA,
        pltpu.SemaphoreType.DMA,
        pltpu.SemaphoreType.DMA,
        pltpu.SemaphoreType.DMA,
    ],
)
def _sc_pass_a(src2_hbm, dst2_hbm, xl_hbm, xr_hbm, ee_hbm, att_hbm,
               lg_hbm, sum_hbm,
               idx_sa, idx_da, xlr0, xrr0, eer0, xlr1, xrr1, eer1,
               lbuf, attb, shared,
               sa0, sb0, sc0, sa1, sb1, sc1):
    cid = lax.axis_index("c")
    sid = lax.axis_index("s")
    wid = sid * NC + cid
    base = wid * EPT

    _zero_shared(shared, lbuf, 16, sid, BC)
    plsc.subcore_barrier()

    pltpu.sync_copy(src2_hbm.at[pl.ds(wid * NCH, NCH)], idx_sa)
    pltpu.sync_copy(dst2_hbm.at[pl.ds(wid * NCH, NCH)], idx_da)
    pltpu.sync_copy(att_hbm, attb)
    lanes = lax.iota(jnp.int32, 16)
    ohs = [(lanes == h).astype(jnp.float32) for h in range(H)]
    c5 = (lanes == H).astype(jnp.float32)

    bufs = ((xlr0, xrr0, eer0, sa0, sb0, sc0),
            (xlr1, xrr1, eer1, sa1, sb1, sc1))

    def fire(j, bs):
        xlr, xrr, eer, sa, sb, sc = bs
        pltpu.async_copy(xl_hbm.at[idx_sa.at[j]], xlr, sa)
        pltpu.async_copy(xr_hbm.at[idx_da.at[j]], xrr, sb)
        pltpu.async_copy(ee_hbm.at[pl.ds(base + j * BC, BC)], eer, sc)

    def drain(bs):
        xlr, xrr, eer, sa, sb, sc = bs
        pltpu.make_async_copy(xl_hbm.at[idx_sa.at[0]], xlr, sa).wait()
        pltpu.make_async_copy(xr_hbm.at[idx_da.at[0]], xrr, sb).wait()
        pltpu.make_async_copy(ee_hbm.at[pl.ds(base, BC)], eer, sc).wait()

    def compute(j, bs):
        xlr, xrr, eer = bs[0], bs[1], bs[2]

        def edge(e, _):
            lv = c5
            for h in range(H):
                acc = None
                for r in (2 * h, 2 * h + 1):
                    sl = pl.ds(r * 16, 16)
                    a = xlr[e, sl] + xrr[e, sl] + eer[e, sl]
                    t = jnp.maximum(a, 0.0) + NEG * jnp.minimum(a, 0.0)
                    p = t * attb[sl]
                    acc = p if acc is None else acc + p
                lv = lv + jnp.sum(acc) * ohs[h]
            lbuf[e, :] = lv
            return 0

        lax.fori_loop(0, BC, edge, 0)
        pltpu.sync_copy(lbuf, lg_hbm.at[pl.ds(base + j * BC, BC)])
        pltpu.sync_copy(lbuf, shared.at[idx_da.at[j]], add=True)

    fire(0, bufs[0])

    def pair(g, _):
        j0 = 2 * g
        drain(bufs[0])
        fire(j0 + 1, bufs[1])
        compute(j0, bufs[0])
        drain(bufs[1])
        fire(j0 + 2, bufs[0])
        compute(j0 + 1, bufs[1])
        return 0

    lax.fori_loop(0, (NCH - 1) // 2, pair, 0)
    drain(bufs[0])
    compute(NCH - 1, bufs[0])
    plsc.subcore_barrier()

    @pl.when(sid == 0)
    def _():
        pltpu.sync_copy(shared, sum_hbm.at[cid])


GRP = 50

@functools.partial(
    pl.kernel,
    out_type=jax.ShapeDtypeStruct((NC, NN, HC + 16), jnp.float32),
    mesh=_mesh,
    compiler_params=pltpu.CompilerParams(use_tc_tiling_on_sc=False,
                                         needs_layout_passes=False),
    scratch_types=[
        pltpu.VMEM((GRP, BCB), jnp.int32),
        pltpu.VMEM((GRP, BCB), jnp.int32),
        pltpu.VMEM((BCB, HC), jnp.float32),
        pltpu.VMEM((BCB, 16), jnp.float32),
        pltpu.VMEM((BCB, 16), jnp.float32),
        pltpu.VMEM((BCB, HC + 16), jnp.float32),
        pltpu.VMEM_SHARED((NN, HC + 16), jnp.float32),
        pltpu.SemaphoreType.DMA,
        pltpu.SemaphoreType.DMA,
        pltpu.SemaphoreType.DMA,
    ],
)
def _sc_pass_b(src2_hbm, dst2_hbm, lg_hbm, mbar_hbm, xl_hbm, out_hbm,
               idx_sa, idx_da, xlr, lr, mr, ob, shared, sem1, sem2, sem3):
    cid = lax.axis_index("c")
    sid = lax.axis_index("s")
    wid = sid * NC + cid
    base = wid * EPT

    _zero_shared(shared, ob, HC + 16, sid, BCB)
    plsc.subcore_barrier()

    lanes = lax.iota(jnp.int32, 16)
    ohs = [(lanes == h).astype(jnp.float32) for h in range(H)]
    msk = (lanes < H).astype(jnp.float32)

    def group(gidx, _):
        grow = wid * NCHB + gidx * GRP
        ci1 = pltpu.async_copy(src2_hbm.at[pl.ds(grow, GRP)], idx_sa, sem1)
        ci2 = pltpu.async_copy(dst2_hbm.at[pl.ds(grow, GRP)], idx_da, sem2)
        ci1.wait()
        ci2.wait()

        def chunk(k, _):
            j = gidx * GRP + k
            off = base + j * BCB
            cp1 = pltpu.async_copy(xl_hbm.at[idx_sa.at[k]], xlr, sem1)
            cp2 = pltpu.async_copy(mbar_hbm.at[idx_da.at[k]], mr, sem2)
            cp3 = pltpu.async_copy(lg_hbm.at[pl.ds(off, BCB)], lr, sem3)
            cp1.wait()
            cp2.wait()
            cp3.wait()

            def edge(e, _):
                ev = jnp.exp(lr[e, :] - mr[e, :]) * msk
                ob[e, pl.ds(HC, 16)] = ev
                for h in range(H):
                    sh = jnp.sum(ev * ohs[h])
                    for r in (2 * h, 2 * h + 1):
                        sl = pl.ds(r * 16, 16)
                        ob[e, sl] = xlr[e, sl] * sh
                return 0

            lax.fori_loop(0, BCB, edge, 0)
            pltpu.sync_copy(ob, shared.at[idx_da.at[k]], add=True)
            return 0

        lax.fori_loop(0, GRP, chunk, 0)
        return 0

    lax.fori_loop(0, NCHB // GRP, group, 0)
    plsc.subcore_barrier()

    @pl.when(sid == 0)
    def _():
        pltpu.sync_copy(shared, out_hbm.at[cid])


def _mm_bias(xarr, W, b, bm):
    M, K = xarr.shape
    Nout = W.shape[1]

    def body(x_ref, w_ref, b_ref, o_ref):
        o_ref[...] = jnp.dot(x_ref[...], w_ref[...],
                             preferred_element_type=jnp.float32) + b_ref[...]

    return pl.pallas_call(
        body,
        grid=(M // bm,),
        in_specs=[
            pl.BlockSpec((bm, K), lambda i: (i, 0)),
            pl.BlockSpec((K, Nout), lambda i: (0, 0)),
            pl.BlockSpec((1, Nout), lambda i: (0, 0)),
        ],
        out_specs=pl.BlockSpec((bm, Nout), lambda i: (i, 0)),
        out_shape=jax.ShapeDtypeStruct((M, Nout), jnp.float32),
    )(xarr, W, b.reshape(1, -1))


def _tc_prep(sl0, sl1, es0, es1, xl, xr, We, att_row, bm=400):
    def body(sl0_ref, sl1_ref, es0_ref, es1_ref, xl_ref, xr_ref, we_ref,
             att_ref, seg_ref, mbar_ref, elp_ref):
        slt = sl0_ref[...] + sl1_ref[...]
        est = es0_ref[...] + es1_ref[...]
        cnt = slt[:, H:H + 1]
        loop_attr = est / jnp.maximum(cnt, 1.0)
        lee = jnp.dot(loop_attr, we_ref[...], preferred_element_type=jnp.float32)
        a = xl_ref[...] + xr_ref[...] + lee
        t = jnp.maximum(a, 0.0) + NEG * jnp.minimum(a, 0.0)
        ll = jnp.dot(t * att_ref[...], seg_ref[...],
                     preferred_element_type=jnp.float32)
        mbar = (slt + ll) / (cnt + 1.0)
        mbar_ref[...] = mbar
        elp_ref[...] = jnp.exp(ll - mbar)

    n16 = pl.BlockSpec((bm, 16), lambda i: (i, 0))
    return pl.pallas_call(
        body,
        grid=(NN // bm,),
        in_specs=[
            n16, n16, n16, n16,
            pl.BlockSpec((bm, HC), lambda i: (i, 0)),
            pl.BlockSpec((bm, HC), lambda i: (i, 0)),
            pl.BlockSpec((ED, HC), lambda i: (0, 0)),
            pl.BlockSpec((1, HC), lambda i: (0, 0)),
            pl.BlockSpec((HC, 16), lambda i: (0, 0)),
        ],
        out_specs=[n16, n16],
        out_shape=[
            jax.ShapeDtypeStruct((NN, 16), jnp.float32),
            jax.ShapeDtypeStruct((NN, 16), jnp.float32),
        ],
    )(sl0, sl1, es0, es1, xl, xr, We, att_row, SEG16)


def _tc_epilogue(a0, a1, xl, elp, bias_row, bm=400):
    def body(a0_ref, a1_ref, xl_ref, elp_ref, b_ref, segt_ref, o_ref):
        acc = a0_ref[...] + a1_ref[...]
        elp = elp_ref[...]
        s16 = acc[:, HC:HC + 16] + elp
        den = jnp.dot(s16, segt_ref[...], preferred_element_type=jnp.float32)
        eexp = jnp.dot(elp, segt_ref[...], preferred_element_type=jnp.float32)
        num = acc[:, 0:HC] + xl_ref[...] * eexp
        o_ref[...] = jnp.maximum(num / (den + 1e-16) + b_ref[...], 0.0)

    return pl.pallas_call(
        body,
        grid=(NN // bm,),
        in_specs=[
            pl.BlockSpec((bm, HC + 16), lambda i: (i, 0)),
            pl.BlockSpec((bm, HC + 16), lambda i: (i, 0)),
            pl.BlockSpec((bm, HC), lambda i: (i, 0)),
            pl.BlockSpec((bm, 16), lambda i: (i, 0)),
            pl.BlockSpec((1, HC), lambda i: (0, 0)),
            pl.BlockSpec((16, HC), lambda i: (0, 0)),
        ],
        out_specs=pl.BlockSpec((bm, HC), lambda i: (i, 0)),
        out_shape=jax.ShapeDtypeStruct((NN, HC), jnp.float32),
    )(a0, a1, xl, elp, bias_row, SEGT16)


def _tc_final(h, Wo1, bo1, Wo2, bo2, bm=400):
    def body(h_ref, w1_ref, b1_ref, w2_ref, b2_ref, o_ref):
        t = jnp.dot(h_ref[...], w1_ref[...],
                    preferred_element_type=jnp.float32) + b1_ref[...]
        o_ref[...] = jnp.dot(t, w2_ref[...],
                             preferred_element_type=jnp.float32) + b2_ref[...]

    return pl.pallas_call(
        body,
        grid=(NN // bm,),
        in_specs=[
            pl.BlockSpec((bm, HC), lambda i: (i, 0)),
            pl.BlockSpec((HC, C), lambda i: (0, 0)),
            pl.BlockSpec((1, C), lambda i: (0, 0)),
            pl.BlockSpec((C, 64), lambda i: (0, 0)),
            pl.BlockSpec((1, 64), lambda i: (0, 0)),
        ],
        out_specs=pl.BlockSpec((bm, 64), lambda i: (i, 0)),
        out_shape=jax.ShapeDtypeStruct((NN, 64), jnp.float32),
    )(h, Wo1, bo1.reshape(1, -1), Wo2, bo2.reshape(1, -1))


def _conv(xin, src2, dst2, src2b, dst2b, edge_attr, esum_p,
          Wl, bl, Wr, br, We, att, bias):
    xl = _mm_bias(xin, Wl, bl, 400)
    xr = _mm_bias(xin, Wr, br, 400)
    ee = _mm_bias(edge_attr, We, jnp.zeros((HC,), jnp.float32), 640)
    lg, slp = _sc_pass_a(src2, dst2, xl, xr, ee, att.reshape(-1))
    mbar, elp = _tc_prep(slp[0], slp[1], esum_p[0], esum_p[1], xl, xr,
                         We, att.reshape(1, -1))
    accp = _sc_pass_b(src2b, dst2b, lg, mbar, xl)
    return _tc_epilogue(accp[0], accp[1], xl, elp, bias.reshape(1, -1))


def kernel(x, edge_index, edge_attr,
           Wl1, bl1, Wr1, br1, We1, att1, bias1,
           Wl2, bl2, Wr2, br2, We2, att2, bias2,
           Wo1, bo1, Wo2, bo2):
    src = edge_index[0]
    dst = edge_index[1]
    src2 = src.reshape(EE // BC, BC)
    dst2 = dst.reshape(EE // BC, BC)
    src2b = src.reshape(EE // BCB, BCB)
    dst2b = dst.reshape(EE // BCB, BCB)
    esum_p = _sc_esum(dst, edge_attr)
    h1 = _conv(x, src2, dst2, src2b, dst2b, edge_attr, esum_p,
               Wl1, bl1, Wr1, br1, We1, att1, bias1)
    h2 = _conv(h1, src2, dst2, src2b, dst2b, edge_attr, esum_p,
               Wl2, bl2, Wr2, br2, We2, att2, bias2)
    return _tc_final(h2, Wo1, bo1, Wo2, bo2)


# speedup vs baseline: 22.0698x; 1.0121x over previous
import functools

import jax
import jax.numpy as jnp
import numpy as np
from jax import lax
from jax.experimental import pallas as pl
from jax.experimental.pallas import tpu as pltpu
from jax.experimental.pallas import tpu_sc as plsc

NN = 10000
EE = 320000
H = 5
C = 32
HC = 160
ED = 16
NEG = 0.2

NC = 2
NS = 16
NW = NC * NS
EPT = EE // NW
BC = 80
NCH = EPT // BC
BCB = 40
NCHB = EPT // BCB

_seg = np.zeros((HC, 16), np.float32)
for _h in range(H):
    _seg[_h * C:(_h + 1) * C, _h] = 1.0
SEG16 = _seg
SEGT16 = np.ascontiguousarray(_seg.T)

_mesh = plsc.VectorSubcoreMesh(core_axis_name="c", subcore_axis_name="s")


def _zero_rows(buf, nrows, width):
    zv = jnp.zeros((16,), jnp.float32)

    def body(i, _):
        for r in range(width // 16):
            buf[i, pl.ds(r * 16, 16)] = zv
        return 0

    lax.fori_loop(0, nrows, body, 0)


def _zero_shared(shared, zbuf, width, sid, bc):
    _zero_rows(zbuf, bc, width)

    def body(j, _):
        @pl.when(j % NS == sid)
        def _():
            pltpu.sync_copy(zbuf, shared.at[pl.ds(j * bc, bc)])
        return 0

    lax.fori_loop(0, NN // bc, body, 0)


@functools.partial(
    pl.kernel,
    out_type=jax.ShapeDtypeStruct((NC, NN, ED), jnp.float32),
    mesh=_mesh,
    compiler_params=pltpu.CompilerParams(use_tc_tiling_on_sc=False,
                                         needs_layout_passes=False),
    scratch_types=[
        pltpu.VMEM((BC,), jnp.int32),
        pltpu.VMEM((BC, ED), jnp.float32),
        pltpu.VMEM_SHARED((NN, ED), jnp.float32),
    ],
)
def _sc_esum(dst_hbm, ea_hbm, out_hbm, idx_d, arows, shared):
    cid = lax.axis_index("c")
    sid = lax.axis_index("s")
    wid = sid * NC + cid
    base = wid * EPT

    _zero_shared(shared, arows, ED, sid, BC)
    plsc.subcore_barrier()

    def chunk(j, _):
        off = base + j * BC
        pltpu.sync_copy(dst_hbm.at[pl.ds(off, BC)], idx_d)
        pltpu.sync_copy(ea_hbm.at[pl.ds(off, BC)], arows)
        pltpu.sync_copy(arows, shared.at[idx_d], add=True)
        return 0

    lax.fori_loop(0, NCH, chunk, 0)
    plsc.subcore_barrier()

    @pl.when(sid == 0)
    def _():
        pltpu.sync_copy(shared, out_hbm.at[cid])


@functools.partial(
    pl.kernel,
    out_type=(
        jax.ShapeDtypeStruct((EE, 16), jnp.float32),
        jax.ShapeDtypeStruct((NC, NN, 16), jnp.float32),
    ),
    mesh=_mesh,
    compiler_params=pltpu.CompilerParams(use_tc_tiling_on_sc=False,
                                         needs_layout_passes=False),
    scratch_types=[
        pltpu.VMEM((NCH, BC), jnp.int32),
        pltpu.VMEM((NCH, BC), jnp.int32),
        pltpu.VMEM((BC, HC), jnp.float32),
        pltpu.VMEM((BC, HC), jnp.float32),
        pltpu.VMEM((BC, HC), jnp.float32),
        pltpu.VMEM((BC, HC), jnp.float32),
        pltpu.VMEM((BC, HC), jnp.float32),
        pltpu.VMEM((BC, HC), jnp.float32),
        pltpu.VMEM((BC, 16), jnp.float32),
        pltpu.VMEM((HC,), jnp.float32),
        pltpu.VMEM_SHARED((NN, 16), jnp.float32),
        pltpu.SemaphoreType.DMA,
        pltpu.SemaphoreType.DMA,
        pltpu.SemaphoreType.DMA,
        pltpu.SemaphoreType.DMA,
        pltpu.SemaphoreType.DMA,
        pltpu.SemaphoreType.DMA,
    ],
)
def _sc_pass_a(src2_hbm, dst2_hbm, xl_hbm, xr_hbm, ee_hbm, att_hbm,
               lg_hbm, sum_hbm,
               idx_sa, idx_da, xlr0, xrr0, eer0, xlr1, xrr1, eer1,
               lbuf, attb, shared,
               sa0, sb0, sc0, sa1, sb1, sc1):
    cid = lax.axis_index("c")
    sid = lax.axis_index("s")
    wid = sid * NC + cid
    base = wid * EPT

    _zero_shared(shared, lbuf, 16, sid, BC)
    plsc.subcore_barrier()

    pltpu.sync_copy(src2_hbm.at[pl.ds(wid * NCH, NCH)], idx_sa)
    pltpu.sync_copy(dst2_hbm.at[pl.ds(wid * NCH, NCH)], idx_da)
    pltpu.sync_copy(att_hbm, attb)
    lanes = lax.iota(jnp.int32, 16)
    ohs = [(lanes == h).astype(jnp.float32) for h in range(H)]
    c5 = (lanes == H).astype(jnp.float32)

    bufs = ((xlr0, xrr0, eer0, sa0, sb0, sc0),
            (xlr1, xrr1, eer1, sa1, sb1, sc1))

    def fire(j, bs):
        xlr, xrr, eer, sa, sb, sc = bs
        pltpu.async_copy(xl_hbm.at[idx_sa.at[j]], xlr, sa)
        pltpu.async_copy(xr_hbm.at[idx_da.at[j]], xrr, sb)
        pltpu.async_copy(ee_hbm.at[pl.ds(base + j * BC, BC)], eer, sc)

    def drain(bs):
        xlr, xrr, eer, sa, sb, sc = bs
        pltpu.make_async_copy(xl_hbm.at[idx_sa.at[0]], xlr, sa).wait()
        pltpu.make_async_copy(xr_hbm.at[idx_da.at[0]], xrr, sb).wait()
        pltpu.make_async_copy(ee_hbm.at[pl.ds(base, BC)], eer, sc).wait()

    def compute(j, bs):
        xlr, xrr, eer = bs[0], bs[1], bs[2]

        def edge(e, _):
            lv = c5
            for h in range(H):
                acc = None
                for r in (2 * h, 2 * h + 1):
                    sl = pl.ds(r * 16, 16)
                    a = xlr[e, sl] + xrr[e, sl] + eer[e, sl]
                    t = jnp.maximum(a, 0.0) + NEG * jnp.minimum(a, 0.0)
                    p = t * attb[sl]
                    acc = p if acc is None else acc + p
                lv = lv + jnp.sum(acc) * ohs[h]
            lbuf[e, :] = lv
            return 0

        lax.fori_loop(0, BC, edge, 0)
        pltpu.sync_copy(lbuf, lg_hbm.at[pl.ds(base + j * BC, BC)])
        pltpu.sync_copy(lbuf, shared.at[idx_da.at[j]], add=True)

    fire(0, bufs[0])

    def pair(g, _):
        j0 = 2 * g
        drain(bufs[0])
        fire(j0 + 1, bufs[1])
        compute(j0, bufs[0])
        drain(bufs[1])
        fire(j0 + 2, bufs[0])
        compute(j0 + 1, bufs[1])
        return 0

    lax.fori_loop(0, (NCH - 1) // 2, pair, 0)
    drain(bufs[0])
    compute(NCH - 1, bufs[0])
    plsc.subcore_barrier()

    @pl.when(sid == 0)
    def _():
        pltpu.sync_copy(shared, sum_hbm.at[cid])


GRP = 50

@functools.partial(
    pl.kernel,
    out_type=jax.ShapeDtypeStruct((NC, NN, HC + 16), jnp.float32),
    mesh=_mesh,
    compiler_params=pltpu.CompilerParams(use_tc_tiling_on_sc=False,
                                         needs_layout_passes=False),
    scratch_types=[
        pltpu.VMEM((GRP, BCB), jnp.int32),
        pltpu.VMEM((GRP, BCB), jnp.int32),
        pltpu.VMEM((BCB, HC), jnp.float32),
        pltpu.VMEM((BCB, 16), jnp.float32),
        pltpu.VMEM((BCB, 16), jnp.float32),
        pltpu.VMEM((BCB, HC + 16), jnp.float32),
        pltpu.VMEM_SHARED((NN, HC + 16), jnp.float32),
        pltpu.SemaphoreType.DMA,
        pltpu.SemaphoreType.DMA,
        pltpu.SemaphoreType.DMA,
    ],
)
def _sc_pass_b(src2_hbm, dst2_hbm, lg_hbm, mbar_hbm, xl_hbm, out_hbm,
               idx_sa, idx_da, xlr, lr, mr, ob, shared, sem1, sem2, sem3):
    cid = lax.axis_index("c")
    sid = lax.axis_index("s")
    wid = sid * NC + cid
    base = wid * EPT

    _zero_shared(shared, ob, HC + 16, sid, BCB)
    plsc.subcore_barrier()

    lanes = lax.iota(jnp.int32, 16)
    ohs = [(lanes == h).astype(jnp.float32) for h in range(H)]
    msk = (lanes < H).astype(jnp.float32)

    def group(gidx, _):
        grow = wid * NCHB + gidx * GRP
        ci1 = pltpu.async_copy(src2_hbm.at[pl.ds(grow, GRP)], idx_sa, sem1)
        ci2 = pltpu.async_copy(dst2_hbm.at[pl.ds(grow, GRP)], idx_da, sem2)
        ci1.wait()
        ci2.wait()

        def chunk(k, _):
            j = gidx * GRP + k
            off = base + j * BCB
            cp1 = pltpu.async_copy(xl_hbm.at[idx_sa.at[k]], xlr, sem1)
            cp2 = pltpu.async_copy(mbar_hbm.at[idx_da.at[k]], mr, sem2)
            cp3 = pltpu.async_copy(lg_hbm.at[pl.ds(off, BCB)], lr, sem3)
            cp1.wait()
            cp2.wait()
            cp3.wait()

            def edge(e, _):
                ev = jnp.exp(lr[e, :] - mr[e, :]) * msk
                ob[e, pl.ds(HC, 16)] = ev
                for h in range(H):
                    sh = jnp.sum(ev * ohs[h])
                    for r in (2 * h, 2 * h + 1):
                        sl = pl.ds(r * 16, 16)
                        ob[e, sl] = xlr[e, sl] * sh
                return 0

            lax.fori_loop(0, BCB, edge, 0)
            pltpu.sync_copy(ob, shared.at[idx_da.at[k]], add=True)
            return 0

        lax.fori_loop(0, GRP, chunk, 0)
        return 0

    lax.fori_loop(0, NCHB // GRP, group, 0)
    plsc.subcore_barrier()

    @pl.when(sid == 0)
    def _():
        pltpu.sync_copy(shared, out_hbm.at[cid])


def _mm_bias2(xarr, Wa, ba, Wb, bb, bm=400):
    M, K = xarr.shape
    Nout = Wa.shape[1]

    def body(x_ref, wa_ref, ba_ref, wb_ref, bb_ref, oa_ref, ob_ref):
        xv = x_ref[...]
        oa_ref[...] = jnp.dot(xv, wa_ref[...],
                              preferred_element_type=jnp.float32) + ba_ref[...]
        ob_ref[...] = jnp.dot(xv, wb_ref[...],
                              preferred_element_type=jnp.float32) + bb_ref[...]

    return pl.pallas_call(
        body,
        grid=(M // bm,),
        in_specs=[
            pl.BlockSpec((bm, K), lambda i: (i, 0)),
            pl.BlockSpec((K, Nout), lambda i: (0, 0)),
            pl.BlockSpec((1, Nout), lambda i: (0, 0)),
            pl.BlockSpec((K, Nout), lambda i: (0, 0)),
            pl.BlockSpec((1, Nout), lambda i: (0, 0)),
        ],
        out_specs=[pl.BlockSpec((bm, Nout), lambda i: (i, 0)),
                   pl.BlockSpec((bm, Nout), lambda i: (i, 0))],
        out_shape=[jax.ShapeDtypeStruct((M, Nout), jnp.float32),
                   jax.ShapeDtypeStruct((M, Nout), jnp.float32)],
    )(xarr, Wa, ba.reshape(1, -1), Wb, bb.reshape(1, -1))


def _mm_bias(xarr, W, b, bm):
    M, K = xarr.shape
    Nout = W.shape[1]

    def body(x_ref, w_ref, b_ref, o_ref):
        o_ref[...] = jnp.dot(x_ref[...], w_ref[...],
                             preferred_element_type=jnp.float32) + b_ref[...]

    return pl.pallas_call(
        body,
        grid=(M // bm,),
        in_specs=[
            pl.BlockSpec((bm, K), lambda i: (i, 0)),
            pl.BlockSpec((K, Nout), lambda i: (0, 0)),
            pl.BlockSpec((1, Nout), lambda i: (0, 0)),
        ],
        out_specs=pl.BlockSpec((bm, Nout), lambda i: (i, 0)),
        out_shape=jax.ShapeDtypeStruct((M, Nout), jnp.float32),
    )(xarr, W, b.reshape(1, -1))


def _tc_prep(sl0, sl1, es0, es1, xl, xr, We, att_row, bm=400):
    def body(sl0_ref, sl1_ref, es0_ref, es1_ref, xl_ref, xr_ref, we_ref,
             att_ref, seg_ref, mbar_ref, elp_ref):
        slt = sl0_ref[...] + sl1_ref[...]
        est = es0_ref[...] + es1_ref[...]
        cnt = slt[:, H:H + 1]
        loop_attr = est / jnp.maximum(cnt, 1.0)
        lee = jnp.dot(loop_attr, we_ref[...], preferred_element_type=jnp.float32)
        a = xl_ref[...] + xr_ref[...] + lee
        t = jnp.maximum(a, 0.0) + NEG * jnp.minimum(a, 0.0)
        ll = jnp.dot(t * att_ref[...], seg_ref[...],
                     preferred_element_type=jnp.float32)
        mbar = (slt + ll) / (cnt + 1.0)
        mbar_ref[...] = mbar
        elp_ref[...] = jnp.exp(ll - mbar)

    n16 = pl.BlockSpec((bm, 16), lambda i: (i, 0))
    return pl.pallas_call(
        body,
        grid=(NN // bm,),
        in_specs=[
            n16, n16, n16, n16,
            pl.BlockSpec((bm, HC), lambda i: (i, 0)),
            pl.BlockSpec((bm, HC), lambda i: (i, 0)),
            pl.BlockSpec((ED, HC), lambda i: (0, 0)),
            pl.BlockSpec((1, HC), lambda i: (0, 0)),
            pl.BlockSpec((HC, 16), lambda i: (0, 0)),
        ],
        out_specs=[n16, n16],
        out_shape=[
            jax.ShapeDtypeStruct((NN, 16), jnp.float32),
            jax.ShapeDtypeStruct((NN, 16), jnp.float32),
        ],
    )(sl0, sl1, es0, es1, xl, xr, We, att_row, SEG16)


def _tc_epilogue(a0, a1, xl, elp, bias_row, bm=400):
    def body(a0_ref, a1_ref, xl_ref, elp_ref, b_ref, segt_ref, o_ref):
        acc = a0_ref[...] + a1_ref[...]
        elp = elp_ref[...]
        s16 = acc[:, HC:HC + 16] + elp
        den = jnp.dot(s16, segt_ref[...], preferred_element_type=jnp.float32)
        eexp = jnp.dot(elp, segt_ref[...], preferred_element_type=jnp.float32)
        num = acc[:, 0:HC] + xl_ref[...] * eexp
        o_ref[...] = jnp.maximum(num / (den + 1e-16) + b_ref[...], 0.0)

    return pl.pallas_call(
        body,
        grid=(NN // bm,),
        in_specs=[
            pl.BlockSpec((bm, HC + 16), lambda i: (i, 0)),
            pl.BlockSpec((bm, HC + 16), lambda i: (i, 0)),
            pl.BlockSpec((bm, HC), lambda i: (i, 0)),
            pl.BlockSpec((bm, 16), lambda i: (i, 0)),
            pl.BlockSpec((1, HC), lambda i: (0, 0)),
            pl.BlockSpec((16, HC), lambda i: (0, 0)),
        ],
        out_specs=pl.BlockSpec((bm, HC), lambda i: (i, 0)),
        out_shape=jax.ShapeDtypeStruct((NN, HC), jnp.float32),
    )(a0, a1, xl, elp, bias_row, SEGT16)


def _tc_epi_final(a0, a1, xl, elp, bias_row, Wo1, bo1, Wo2, bo2, bm=400):
    def body(a0_ref, a1_ref, xl_ref, elp_ref, b_ref, segt_ref,
             w1_ref, b1_ref, w2_ref, b2_ref, o_ref):
        acc = a0_ref[...] + a1_ref[...]
        elp = elp_ref[...]
        s16 = acc[:, HC:HC + 16] + elp
        den = jnp.dot(s16, segt_ref[...], preferred_element_type=jnp.float32)
        eexp = jnp.dot(elp, segt_ref[...], preferred_element_type=jnp.float32)
        num = acc[:, 0:HC] + xl_ref[...] * eexp
        h = jnp.maximum(num / (den + 1e-16) + b_ref[...], 0.0)
        t = jnp.dot(h, w1_ref[...],
                    preferred_element_type=jnp.float32) + b1_ref[...]
        o_ref[...] = jnp.dot(t, w2_ref[...],
                             preferred_element_type=jnp.float32) + b2_ref[...]

    return pl.pallas_call(
        body,
        grid=(NN // bm,),
        in_specs=[
            pl.BlockSpec((bm, HC + 16), lambda i: (i, 0)),
            pl.BlockSpec((bm, HC + 16), lambda i: (i, 0)),
            pl.BlockSpec((bm, HC), lambda i: (i, 0)),
            pl.BlockSpec((bm, 16), lambda i: (i, 0)),
            pl.BlockSpec((1, HC), lambda i: (0, 0)),
            pl.BlockSpec((16, HC), lambda i: (0, 0)),
            pl.BlockSpec((HC, C), lambda i: (0, 0)),
            pl.BlockSpec((1, C), lambda i: (0, 0)),
            pl.BlockSpec((C, 64), lambda i: (0, 0)),
            pl.BlockSpec((1, 64), lambda i: (0, 0)),
        ],
        out_specs=pl.BlockSpec((bm, 64), lambda i: (i, 0)),
        out_shape=jax.ShapeDtypeStruct((NN, 64), jnp.float32),
    )(a0, a1, xl, elp, bias_row, SEGT16,
      Wo1, bo1.reshape(1, -1), Wo2, bo2.reshape(1, -1))


def _tc_final(h, Wo1, bo1, Wo2, bo2, bm=400):
    def body(h_ref, w1_ref, b1_ref, w2_ref, b2_ref, o_ref):
        t = jnp.dot(h_ref[...], w1_ref[...],
                    preferred_element_type=jnp.float32) + b1_ref[...]
        o_ref[...] = jnp.dot(t, w2_ref[...],
                             preferred_element_type=jnp.float32) + b2_ref[...]

    return pl.pallas_call(
        body,
        grid=(NN // bm,),
        in_specs=[
            pl.BlockSpec((bm, HC), lambda i: (i, 0)),
            pl.BlockSpec((HC, C), lambda i: (0, 0)),
            pl.BlockSpec((1, C), lambda i: (0, 0)),
            pl.BlockSpec((C, 64), lambda i: (0, 0)),
            pl.BlockSpec((1, 64), lambda i: (0, 0)),
        ],
        out_specs=pl.BlockSpec((bm, 64), lambda i: (i, 0)),
        out_shape=jax.ShapeDtypeStruct((NN, 64), jnp.float32),
    )(h, Wo1, bo1.reshape(1, -1), Wo2, bo2.reshape(1, -1))


def _conv(xin, src2, dst2, src2b, dst2b, edge_attr, esum_p,
          Wl, bl, Wr, br, We, att, bias, fin=None):
    xl, xr = _mm_bias2(xin, Wl, bl, Wr, br)
    ee = _mm_bias(edge_attr, We, jnp.zeros((HC,), jnp.float32), 640)
    lg, slp = _sc_pass_a(src2, dst2, xl, xr, ee, att.reshape(-1))
    mbar, elp = _tc_prep(slp[0], slp[1], esum_p[0], esum_p[1], xl, xr,
                         We, att.reshape(1, -1))
    accp = _sc_pass_b(src2b, dst2b, lg, mbar, xl)
    if fin is None:
        return _tc_epilogue(accp[0], accp[1], xl, elp, bias.reshape(1, -1))
    return _tc_epi_final(accp[0], accp[1], xl, elp, bias.reshape(1, -1), *fin)


def kernel(x, edge_index, edge_attr,
           Wl1, bl1, Wr1, br1, We1, att1, bias1,
           Wl2, bl2, Wr2, br2, We2, att2, bias2,
           Wo1, bo1, Wo2, bo2):
    src = edge_index[0]
    dst = edge_index[1]
    src2 = src.reshape(EE // BC, BC)
    dst2 = dst.reshape(EE // BC, BC)
    src2b = src.reshape(EE // BCB, BCB)
    dst2b = dst.reshape(EE // BCB, BCB)
    esum_p = _sc_esum(dst, edge_attr)
    h1 = _conv(x, src2, dst2, src2b, dst2b, edge_attr, esum_p,
               Wl1, bl1, Wr1, br1, We1, att1, bias1)
    return _conv(h1, src2, dst2, src2b, dst2b, edge_attr, esum_p,
                 Wl2, bl2, Wr2, br2, We2, att2, bias2,
                 fin=(Wo1, bo1, Wo2, bo2))


# speedup vs baseline: 22.2535x; 1.0083x over previous
import functools

import jax
import jax.numpy as jnp
import numpy as np
from jax import lax
from jax.experimental import pallas as pl
from jax.experimental.pallas import tpu as pltpu
from jax.experimental.pallas import tpu_sc as plsc

NN = 10000
EE = 320000
H = 5
C = 32
HC = 160
ED = 16
NEG = 0.2

NC = 2
NS = 16
NW = NC * NS
EPT = EE // NW
BC = 80
NCH = EPT // BC
BCB = 40
NCHB = EPT // BCB

_seg = np.zeros((HC, 16), np.float32)
for _h in range(H):
    _seg[_h * C:(_h + 1) * C, _h] = 1.0
SEG16 = _seg
SEGT16 = np.ascontiguousarray(_seg.T)

_mesh = plsc.VectorSubcoreMesh(core_axis_name="c", subcore_axis_name="s")


def _zero_rows(buf, nrows, width):
    zv = jnp.zeros((16,), jnp.float32)

    def body(i, _):
        for r in range(width // 16):
            buf[i, pl.ds(r * 16, 16)] = zv
        return 0

    lax.fori_loop(0, nrows, body, 0)


def _zero_shared(shared, zbuf, width, sid, bc):
    _zero_rows(zbuf, bc, width)

    def body(j, _):
        @pl.when(j % NS == sid)
        def _():
            pltpu.sync_copy(zbuf, shared.at[pl.ds(j * bc, bc)])
        return 0

    lax.fori_loop(0, NN // bc, body, 0)


@functools.partial(
    pl.kernel,
    out_type=jax.ShapeDtypeStruct((NC, NN, ED), jnp.float32),
    mesh=_mesh,
    compiler_params=pltpu.CompilerParams(use_tc_tiling_on_sc=False,
                                         needs_layout_passes=False),
    scratch_types=[
        pltpu.VMEM((BC,), jnp.int32),
        pltpu.VMEM((BC, ED), jnp.float32),
        pltpu.VMEM_SHARED((NN, ED), jnp.float32),
    ],
)
def _sc_esum(dst_hbm, ea_hbm, out_hbm, idx_d, arows, shared):
    cid = lax.axis_index("c")
    sid = lax.axis_index("s")
    wid = sid * NC + cid
    base = wid * EPT

    _zero_shared(shared, arows, ED, sid, BC)
    plsc.subcore_barrier()

    def chunk(j, _):
        off = base + j * BC
        pltpu.sync_copy(dst_hbm.at[pl.ds(off, BC)], idx_d)
        pltpu.sync_copy(ea_hbm.at[pl.ds(off, BC)], arows)
        pltpu.sync_copy(arows, shared.at[idx_d], add=True)
        return 0

    lax.fori_loop(0, NCH, chunk, 0)
    plsc.subcore_barrier()

    @pl.when(sid == 0)
    def _():
        pltpu.sync_copy(shared, out_hbm.at[cid])


@functools.partial(
    pl.kernel,
    out_type=(
        jax.ShapeDtypeStruct((EE, 16), jnp.float32),
        jax.ShapeDtypeStruct((NC, NN, 16), jnp.float32),
    ),
    mesh=_mesh,
    compiler_params=pltpu.CompilerParams(use_tc_tiling_on_sc=False,
                                         needs_layout_passes=False),
    scratch_types=[
        pltpu.VMEM((NCH, BC), jnp.int32),
        pltpu.VMEM((NCH, BC), jnp.int32),
        pltpu.VMEM((BC, HC), jnp.float32),
        pltpu.VMEM((BC, HC), jnp.float32),
        pltpu.VMEM((BC, HC), jnp.float32),
        pltpu.VMEM((BC, HC), jnp.float32),
        pltpu.VMEM((BC, HC), jnp.float32),
        pltpu.VMEM((BC, HC), jnp.float32),
        pltpu.VMEM((BC, 16), jnp.float32),
        pltpu.VMEM((HC,), jnp.float32),
        pltpu.VMEM_SHARED((NN, 16), jnp.float32),
        pltpu.SemaphoreType.DMA,
        pltpu.SemaphoreType.DMA,
        pltpu.SemaphoreType.DMA,
        pltpu.SemaphoreType.DMA,
        pltpu.SemaphoreType.DMA,
        pltpu.SemaphoreType.DMA,
    ],
)
def _sc_pass_a(src2_hbm, dst2_hbm, xl_hbm, xr_hbm, ee_hbm, att_hbm,
               lg_hbm, sum_hbm,
               idx_sa, idx_da, xlr0, xrr0, eer0, xlr1, xrr1, eer1,
               lbuf, attb, shared,
               sa0, sb0, sc0, sa1, sb1, sc1):
    cid = lax.axis_index("c")
    sid = lax.axis_index("s")
    wid = sid * NC + cid
    base = wid * EPT

    _zero_shared(shared, lbuf, 16, sid, BC)
    plsc.subcore_barrier()

    pltpu.sync_copy(src2_hbm.at[pl.ds(wid * NCH, NCH)], idx_sa)
    pltpu.sync_copy(dst2_hbm.at[pl.ds(wid * NCH, NCH)], idx_da)
    pltpu.sync_copy(att_hbm, attb)
    lanes = lax.iota(jnp.int32, 16)
    ohs = [(lanes == h).astype(jnp.float32) for h in range(H)]
    c5 = (lanes == H).astype(jnp.float32)
    av = [attb[pl.ds(r * 16, 16)] for r in range(2 * H)]

    bufs = ((xlr0, xrr0, eer0, sa0, sb0, sc0),
            (xlr1, xrr1, eer1, sa1, sb1, sc1))

    def fire(j, bs):
        xlr, xrr, eer, sa, sb, sc = bs
        pltpu.async_copy(xl_hbm.at[idx_sa.at[j]], xlr, sa)
        pltpu.async_copy(xr_hbm.at[idx_da.at[j]], xrr, sb)
        pltpu.async_copy(ee_hbm.at[pl.ds(base + j * BC, BC)], eer, sc)

    def drain(bs):
        xlr, xrr, eer, sa, sb, sc = bs
        pltpu.make_async_copy(xl_hbm.at[idx_sa.at[0]], xlr, sa).wait()
        pltpu.make_async_copy(xr_hbm.at[idx_da.at[0]], xrr, sb).wait()
        pltpu.make_async_copy(ee_hbm.at[pl.ds(base, BC)], eer, sc).wait()

    def compute(j, bs):
        xlr, xrr, eer = bs[0], bs[1], bs[2]

        def edge(e, _):
            lv = c5
            for h in range(H):
                acc = None
                for r in (2 * h, 2 * h + 1):
                    sl = pl.ds(r * 16, 16)
                    a = xlr[e, sl] + xrr[e, sl] + eer[e, sl]
                    t = jnp.maximum(a, 0.0) + NEG * jnp.minimum(a, 0.0)
                    p = t * av[r]
                    acc = p if acc is None else acc + p
                lv = lv + jnp.sum(acc) * ohs[h]
            lbuf[e, :] = lv
            return 0

        lax.fori_loop(0, BC, edge, 0)
        pltpu.sync_copy(lbuf, lg_hbm.at[pl.ds(base + j * BC, BC)])
        pltpu.sync_copy(lbuf, shared.at[idx_da.at[j]], add=True)

    fire(0, bufs[0])

    def pair(g, _):
        j0 = 2 * g
        drain(bufs[0])
        fire(j0 + 1, bufs[1])
        compute(j0, bufs[0])
        drain(bufs[1])
        fire(j0 + 2, bufs[0])
        compute(j0 + 1, bufs[1])
        return 0

    lax.fori_loop(0, (NCH - 1) // 2, pair, 0)
    drain(bufs[0])
    compute(NCH - 1, bufs[0])
    plsc.subcore_barrier()

    @pl.when(sid == 0)
    def _():
        pltpu.sync_copy(shared, sum_hbm.at[cid])


GRP = 50

@functools.partial(
    pl.kernel,
    out_type=jax.ShapeDtypeStruct((NC, NN, HC + 16), jnp.float32),
    mesh=_mesh,
    compiler_params=pltpu.CompilerParams(use_tc_tiling_on_sc=False,
                                         needs_layout_passes=False),
    scratch_types=[
        pltpu.VMEM((GRP, BCB), jnp.int32),
        pltpu.VMEM((GRP, BCB), jnp.int32),
        pltpu.VMEM((BCB, HC), jnp.float32),
        pltpu.VMEM((BCB, 16), jnp.float32),
        pltpu.VMEM((BCB, 16), jnp.float32),
        pltpu.VMEM((BCB, HC + 16), jnp.float32),
        pltpu.VMEM_SHARED((NN, HC + 16), jnp.float32),
        pltpu.SemaphoreType.DMA,
        pltpu.SemaphoreType.DMA,
        pltpu.SemaphoreType.DMA,
    ],
)
def _sc_pass_b(src2_hbm, dst2_hbm, lg_hbm, mbar_hbm, xl_hbm, out_hbm,
               idx_sa, idx_da, xlr, lr, mr, ob, shared, sem1, sem2, sem3):
    cid = lax.axis_index("c")
    sid = lax.axis_index("s")
    wid = sid * NC + cid
    base = wid * EPT

    _zero_shared(shared, ob, HC + 16, sid, BCB)
    plsc.subcore_barrier()

    lanes = lax.iota(jnp.int32, 16)
    ohs = [(lanes == h).astype(jnp.float32) for h in range(H)]
    msk = (lanes < H).astype(jnp.float32)

    def group(gidx, _):
        grow = wid * NCHB + gidx * GRP
        ci1 = pltpu.async_copy(src2_hbm.at[pl.ds(grow, GRP)], idx_sa, sem1)
        ci2 = pltpu.async_copy(dst2_hbm.at[pl.ds(grow, GRP)], idx_da, sem2)
        ci1.wait()
        ci2.wait()

        def chunk(k, _):
            j = gidx * GRP + k
            off = base + j * BCB
            cp1 = pltpu.async_copy(xl_hbm.at[idx_sa.at[k]], xlr, sem1)
            cp2 = pltpu.async_copy(mbar_hbm.at[idx_da.at[k]], mr, sem2)
            cp3 = pltpu.async_copy(lg_hbm.at[pl.ds(off, BCB)], lr, sem3)
            cp1.wait()
            cp2.wait()
            cp3.wait()

            def edge(e, _):
                ev = jnp.exp(lr[e, :] - mr[e, :]) * msk
                ob[e, pl.ds(HC, 16)] = ev
                for h in range(H):
                    sh = jnp.sum(ev * ohs[h])
                    for r in (2 * h, 2 * h + 1):
                        sl = pl.ds(r * 16, 16)
                        ob[e, sl] = xlr[e, sl] * sh
                return 0

            lax.fori_loop(0, BCB, edge, 0)
            pltpu.sync_copy(ob, shared.at[idx_da.at[k]], add=True)
            return 0

        lax.fori_loop(0, GRP, chunk, 0)
        return 0

    lax.fori_loop(0, NCHB // GRP, group, 0)
    plsc.subcore_barrier()

    @pl.when(sid == 0)
    def _():
        pltpu.sync_copy(shared, out_hbm.at[cid])


def _mm_bias2(xarr, Wa, ba, Wb, bb, bm=400):
    M, K = xarr.shape
    Nout = Wa.shape[1]

    def body(x_ref, wa_ref, ba_ref, wb_ref, bb_ref, oa_ref, ob_ref):
        xv = x_ref[...]
        oa_ref[...] = jnp.dot(xv, wa_ref[...],
                              preferred_element_type=jnp.float32) + ba_ref[...]
        ob_ref[...] = jnp.dot(xv, wb_ref[...],
                              preferred_element_type=jnp.float32) + bb_ref[...]

    return pl.pallas_call(
        body,
        grid=(M // bm,),
        in_specs=[
            pl.BlockSpec((bm, K), lambda i: (i, 0)),
            pl.BlockSpec((K, Nout), lambda i: (0, 0)),
            pl.BlockSpec((1, Nout), lambda i: (0, 0)),
            pl.BlockSpec((K, Nout), lambda i: (0, 0)),
            pl.BlockSpec((1, Nout), lambda i: (0, 0)),
        ],
        out_specs=[pl.BlockSpec((bm, Nout), lambda i: (i, 0)),
                   pl.BlockSpec((bm, Nout), lambda i: (i, 0))],
        out_shape=[jax.ShapeDtypeStruct((M, Nout), jnp.float32),
                   jax.ShapeDtypeStruct((M, Nout), jnp.float32)],
    )(xarr, Wa, ba.reshape(1, -1), Wb, bb.reshape(1, -1))


def _mm_bias(xarr, W, b, bm):
    M, K = xarr.shape
    Nout = W.shape[1]

    def body(x_ref, w_ref, b_ref, o_ref):
        o_ref[...] = jnp.dot(x_ref[...], w_ref[...],
                             preferred_element_type=jnp.float32) + b_ref[...]

    return pl.pallas_call(
        body,
        grid=(M // bm,),
        in_specs=[
            pl.BlockSpec((bm, K), lambda i: (i, 0)),
            pl.BlockSpec((K, Nout), lambda i: (0, 0)),
            pl.BlockSpec((1, Nout), lambda i: (0, 0)),
        ],
        out_specs=pl.BlockSpec((bm, Nout), lambda i: (i, 0)),
        out_shape=jax.ShapeDtypeStruct((M, Nout), jnp.float32),
    )(xarr, W, b.reshape(1, -1))


def _tc_prep(sl0, sl1, es0, es1, xl, xr, We, att_row, bm=400):
    def body(sl0_ref, sl1_ref, es0_ref, es1_ref, xl_ref, xr_ref, we_ref,
             att_ref, seg_ref, mbar_ref, elp_ref):
        slt = sl0_ref[...] + sl1_ref[...]
        est = es0_ref[...] + es1_ref[...]
        cnt = slt[:, H:H + 1]
        loop_attr = est / jnp.maximum(cnt, 1.0)
        lee = jnp.dot(loop_attr, we_ref[...], preferred_element_type=jnp.float32)
        a = xl_ref[...] + xr_ref[...] + lee
        t = jnp.maximum(a, 0.0) + NEG * jnp.minimum(a, 0.0)
        ll = jnp.dot(t * att_ref[...], seg_ref[...],
                     preferred_element_type=jnp.float32)
        mbar = (slt + ll) / (cnt + 1.0)
        mbar_ref[...] = mbar
        elp_ref[...] = jnp.exp(ll - mbar)

    n16 = pl.BlockSpec((bm, 16), lambda i: (i, 0))
    return pl.pallas_call(
        body,
        grid=(NN // bm,),
        in_specs=[
            n16, n16, n16, n16,
            pl.BlockSpec((bm, HC), lambda i: (i, 0)),
            pl.BlockSpec((bm, HC), lambda i: (i, 0)),
            pl.BlockSpec((ED, HC), lambda i: (0, 0)),
            pl.BlockSpec((1, HC), lambda i: (0, 0)),
            pl.BlockSpec((HC, 16), lambda i: (0, 0)),
        ],
        out_specs=[n16, n16],
        out_shape=[
            jax.ShapeDtypeStruct((NN, 16), jnp.float32),
            jax.ShapeDtypeStruct((NN, 16), jnp.float32),
        ],
    )(sl0, sl1, es0, es1, xl, xr, We, att_row, SEG16)


def _tc_epilogue(a0, a1, xl, elp, bias_row, bm=400):
    def body(a0_ref, a1_ref, xl_ref, elp_ref, b_ref, segt_ref, o_ref):
        acc = a0_ref[...] + a1_ref[...]
        elp = elp_ref[...]
        s16 = acc[:, HC:HC + 16] + elp
        den = jnp.dot(s16, segt_ref[...], preferred_element_type=jnp.float32)
        eexp = jnp.dot(elp, segt_ref[...], preferred_element_type=jnp.float32)
        num = acc[:, 0:HC] + xl_ref[...] * eexp
        o_ref[...] = jnp.maximum(num / (den + 1e-16) + b_ref[...], 0.0)

    return pl.pallas_call(
        body,
        grid=(NN // bm,),
        in_specs=[
            pl.BlockSpec((bm, HC + 16), lambda i: (i, 0)),
            pl.BlockSpec((bm, HC + 16), lambda i: (i, 0)),
            pl.BlockSpec((bm, HC), lambda i: (i, 0)),
            pl.BlockSpec((bm, 16), lambda i: (i, 0)),
            pl.BlockSpec((1, HC), lambda i: (0, 0)),
            pl.BlockSpec((16, HC), lambda i: (0, 0)),
        ],
        out_specs=pl.BlockSpec((bm, HC), lambda i: (i, 0)),
        out_shape=jax.ShapeDtypeStruct((NN, HC), jnp.float32),
    )(a0, a1, xl, elp, bias_row, SEGT16)


def _tc_epi_final(a0, a1, xl, elp, bias_row, Wo1, bo1, Wo2, bo2, bm=400):
    def body(a0_ref, a1_ref, xl_ref, elp_ref, b_ref, segt_ref,
             w1_ref, b1_ref, w2_ref, b2_ref, o_ref):
        acc = a0_ref[...] + a1_ref[...]
        elp = elp_ref[...]
        s16 = acc[:, HC:HC + 16] + elp
        den = jnp.dot(s16, segt_ref[...], preferred_element_type=jnp.float32)
        eexp = jnp.dot(elp, segt_ref[...], preferred_element_type=jnp.float32)
        num = acc[:, 0:HC] + xl_ref[...] * eexp
        h = jnp.maximum(num / (den + 1e-16) + b_ref[...], 0.0)
        t = jnp.dot(h, w1_ref[...],
                    preferred_element_type=jnp.float32) + b1_ref[...]
        o_ref[...] = jnp.dot(t, w2_ref[...],
                             preferred_element_type=jnp.float32) + b2_ref[...]

    return pl.pallas_call(
        body,
        grid=(NN // bm,),
        in_specs=[
            pl.BlockSpec((bm, HC + 16), lambda i: (i, 0)),
            pl.BlockSpec((bm, HC + 16), lambda i: (i, 0)),
            pl.BlockSpec((bm, HC), lambda i: (i, 0)),
            pl.BlockSpec((bm, 16), lambda i: (i, 0)),
            pl.BlockSpec((1, HC), lambda i: (0, 0)),
            pl.BlockSpec((16, HC), lambda i: (0, 0)),
            pl.BlockSpec((HC, C), lambda i: (0, 0)),
            pl.BlockSpec((1, C), lambda i: (0, 0)),
            pl.BlockSpec((C, 64), lambda i: (0, 0)),
            pl.BlockSpec((1, 64), lambda i: (0, 0)),
        ],
        out_specs=pl.BlockSpec((bm, 64), lambda i: (i, 0)),
        out_shape=jax.ShapeDtypeStruct((NN, 64), jnp.float32),
    )(a0, a1, xl, elp, bias_row, SEGT16,
      Wo1, bo1.reshape(1, -1), Wo2, bo2.reshape(1, -1))


def _tc_final(h, Wo1, bo1, Wo2, bo2, bm=400):
    def body(h_ref, w1_ref, b1_ref, w2_ref, b2_ref, o_ref):
        t = jnp.dot(h_ref[...], w1_ref[...],
                    preferred_element_type=jnp.float32) + b1_ref[...]
        o_ref[...] = jnp.dot(t, w2_ref[...],
                             preferred_element_type=jnp.float32) + b2_ref[...]

    return pl.pallas_call(
        body,
        grid=(NN // bm,),
        in_specs=[
            pl.BlockSpec((bm, HC), lambda i: (i, 0)),
            pl.BlockSpec((HC, C), lambda i: (0, 0)),
            pl.BlockSpec((1, C), lambda i: (0, 0)),
            pl.BlockSpec((C, 64), lambda i: (0, 0)),
            pl.BlockSpec((1, 64), lambda i: (0, 0)),
        ],
        out_specs=pl.BlockSpec((bm, 64), lambda i: (i, 0)),
        out_shape=jax.ShapeDtypeStruct((NN, 64), jnp.float32),
    )(h, Wo1, bo1.reshape(1, -1), Wo2, bo2.reshape(1, -1))


def _conv(xin, src2, dst2, src2b, dst2b, edge_attr, esum_p,
          Wl, bl, Wr, br, We, att, bias, fin=None):
    xl, xr = _mm_bias2(xin, Wl, bl, Wr, br)
    ee = _mm_bias(edge_attr, We, jnp.zeros((HC,), jnp.float32), 640)
    lg, slp = _sc_pass_a(src2, dst2, xl, xr, ee, att.reshape(-1))
    mbar, elp = _tc_prep(slp[0], slp[1], esum_p[0], esum_p[1], xl, xr,
                         We, att.reshape(1, -1))
    accp = _sc_pass_b(src2b, dst2b, lg, mbar, xl)
    if fin is None:
        return _tc_epilogue(accp[0], accp[1], xl, elp, bias.reshape(1, -1))
    return _tc_epi_final(accp[0], accp[1], xl, elp, bias.reshape(1, -1), *fin)


def kernel(x, edge_index, edge_attr,
           Wl1, bl1, Wr1, br1, We1, att1, bias1,
           Wl2, bl2, Wr2, br2, We2, att2, bias2,
           Wo1, bo1, Wo2, bo2):
    src = edge_index[0]
    dst = edge_index[1]
    src2 = src.reshape(EE // BC, BC)
    dst2 = dst.reshape(EE // BC, BC)
    src2b = src.reshape(EE // BCB, BCB)
    dst2b = dst.reshape(EE // BCB, BCB)
    esum_p = _sc_esum(dst, edge_attr)
    h1 = _conv(x, src2, dst2, src2b, dst2b, edge_attr, esum_p,
               Wl1, bl1, Wr1, br1, We1, att1, bias1)
    return _conv(h1, src2, dst2, src2b, dst2b, edge_attr, esum_p,
                 Wl2, bl2, Wr2, br2, We2, att2, bias2,
                 fin=(Wo1, bo1, Wo2, bo2))
